# baseline (speedup 1.0000x reference)
import functools

import jax
import jax.numpy as jnp
from jax import lax
from jax.experimental import pallas as pl
from jax.experimental.pallas import tpu as pltpu
from jax.experimental.pallas import tpu_sc as plsc

_B = 4096
_S = 200
_D = 64

_NC = 2
_NS = 16
_NW = _NC * _NS
_BLK = _B // _NW

_mesh = plsc.VectorSubcoreMesh(core_axis_name="c", subcore_axis_name="s")


@functools.partial(
    pl.kernel,
    out_type=jax.ShapeDtypeStruct((_S, _D, _B), jnp.float32),
    mesh=_mesh,
    scratch_types=[
        pltpu.VMEM((_S, _BLK), jnp.int32),
        pltpu.VMEM((2, _BLK, _D), jnp.float32),
        pltpu.VMEM((2, _D, _BLK), jnp.float32),
        pltpu.SemaphoreType.DMA,
        pltpu.SemaphoreType.DMA,
        pltpu.SemaphoreType.DMA,
        pltpu.SemaphoreType.DMA,
    ],
    compiler_params=pltpu.CompilerParams(
        use_tc_tiling_on_sc=False, needs_layout_passes=False
    ),
)
def _sc_gather_t(xt_hbm, table_hbm, out_hbm, idx_v, rows_v, tblk_v, sg0, sg1,
                 so0, so1):
    w = lax.axis_index("s") * _NC + lax.axis_index("c")
    b0 = w * _BLK
    col = lax.iota(jnp.int32, 16)
    rowidx = [16 * j + col for j in range(4)]

    pltpu.sync_copy(xt_hbm.at[:, pl.ds(b0, _BLK)], idx_v)

    def gather(s, buf, sem):
        return pltpu.make_async_copy(table_hbm.at[idx_v.at[s]], rows_v.at[buf], sem)

    def transpose(buf):
        rows = rows_v.at[buf]
        tflat = tblk_v.at[buf]

        def row(b, carry):
            bvec = jnp.full((16,), b, jnp.int32)
            for j in range(4):
                v = rows[b, pl.ds(16 * j, 16)]
                plsc.store_scatter(tflat, [rowidx[j], bvec], v)
            return carry

        lax.fori_loop(0, _BLK, row, 0, unroll=2)

    def out_copy(s, buf, sem):
        return pltpu.make_async_copy(
            tblk_v.at[buf],
            out_hbm.at[s, :, pl.ds(b0, _BLK)],
            sem,
        )

    gsems = (sg0, sg1)
    osems = (so0, so1)
    gather(0, 0, sg0).start()

    def pair(i, carry):
        s0 = 2 * i
        for p in (0, 1):
            s = s0 + p
            nxt = gsems[1 - p]

            @pl.when(s + 1 < _S)
            def _():
                gather(s + 1, 1 - p, nxt).start()

            gather(s, p, gsems[p]).wait()

            @pl.when(s >= 2)
            def _():
                out_copy(s - 2, p, osems[p]).wait()

            transpose(p)
            out_copy(s, p, osems[p]).start()
        return carry

    lax.fori_loop(0, _S // 2, pair, 0)
    out_copy(_S - 2, 0, so0).wait()
    out_copy(_S - 1, 1, so1).wait()


def kernel(x, vocab_embed):
    b, s = x.shape
    d = vocab_embed.shape[1]
    xt = x.T
    outt = _sc_gather_t(xt, vocab_embed)
    real = outt.transpose(2, 0, 1)
    omega = 1.0 / (10000.0 ** (jnp.arange(0, d, 2, dtype=jnp.float32) / d))
    angles = omega[None, :] * jnp.arange(s, dtype=jnp.float32)[:, None]
    imag = jnp.repeat(jnp.sin(angles), 2, axis=-1)
    imag = jnp.broadcast_to(imag[None, :, :], (b, s, d))
    return jax.lax.complex(real, imag)


# speedup vs baseline: 1.0600x; 1.0600x over previous
import functools

import jax
import jax.numpy as jnp
from jax import lax
from jax.experimental import pallas as pl
from jax.experimental.pallas import tpu as pltpu
from jax.experimental.pallas import tpu_sc as plsc

_B = 4096
_S = 200
_D = 64

_NC = 2
_NS = 16
_NW = _NC * _NS
_BLK = _B // _NW
_SP = 2
_NU = _S // _SP

_mesh = plsc.VectorSubcoreMesh(core_axis_name="c", subcore_axis_name="s")


@functools.partial(
    pl.kernel,
    out_type=jax.ShapeDtypeStruct((_S, _D, _B), jnp.float32),
    mesh=_mesh,
    scratch_types=[
        pltpu.VMEM((_S, _BLK), jnp.int32),
        pltpu.VMEM((2, _SP * _BLK, _D), jnp.float32),
        pltpu.VMEM((2, _SP, _D, _BLK), jnp.float32),
        pltpu.SemaphoreType.DMA,
        pltpu.SemaphoreType.DMA,
        pltpu.SemaphoreType.DMA,
        pltpu.SemaphoreType.DMA,
    ],
    compiler_params=pltpu.CompilerParams(
        use_tc_tiling_on_sc=False, needs_layout_passes=False
    ),
)
def _sc_gather_t(xt_hbm, table_hbm, out_hbm, idx_v, rows_v, tblk_v, sg0, sg1,
                 so0, so1):
    w = lax.axis_index("s") * _NC + lax.axis_index("c")
    b0 = w * _BLK
    col = lax.iota(jnp.int32, 16)
    rowidx = [16 * j + col for j in range(4)]

    pltpu.sync_copy(xt_hbm.at[:, pl.ds(b0, _BLK)], idx_v)

    def gather_parts(u, buf, sem):
        return [
            pltpu.make_async_copy(
                table_hbm.at[idx_v.at[_SP * u + k]],
                rows_v.at[buf, pl.ds(k * _BLK, _BLK)],
                sem,
            )
            for k in range(_SP)
        ]

    def gather_start(u, buf, sem):
        for c in gather_parts(u, buf, sem):
            c.start()

    def gather_wait(u, buf, sem):
        for c in gather_parts(u, buf, sem):
            c.wait()

    def transpose(buf):
        rows = rows_v.at[buf]
        tdst = tblk_v.at[buf]

        @plsc.parallel_loop(0, _SP * _BLK, unroll=8)
        def row(r):
            sp = r // _BLK
            b = r % _BLK
            bvec = jnp.full((16,), b, jnp.int32)
            for j in range(4):
                v = rows[r, pl.ds(16 * j, 16)]
                plsc.store_scatter(tdst.at[sp], [rowidx[j], bvec], v)

    def out_copy(u, buf, sem):
        return pltpu.make_async_copy(
            tblk_v.at[buf],
            out_hbm.at[pl.ds(u * _SP, _SP), :, pl.ds(b0, _BLK)],
            sem,
        )

    gsems = (sg0, sg1)
    osems = (so0, so1)
    gather_start(0, 0, sg0)

    def pair(i, carry):
        u0 = 2 * i
        for p in (0, 1):
            u = u0 + p
            nxt = gsems[1 - p]

            @pl.when(u + 1 < _NU)
            def _():
                gather_start(u + 1, 1 - p, nxt)

            gather_wait(u, p, gsems[p])

            @pl.when(u >= 2)
            def _():
                out_copy(u - 2, p, osems[p]).wait()

            transpose(p)
            out_copy(u, p, osems[p]).start()
        return carry

    lax.fori_loop(0, _NU // 2, pair, 0)
    out_copy(_NU - 2, 0, so0).wait()
    out_copy(_NU - 1, 1, so1).wait()


def kernel(x, vocab_embed):
    b, s = x.shape
    d = vocab_embed.shape[1]
    xt = x.T
    outt = _sc_gather_t(xt, vocab_embed)
    real = outt.transpose(2, 0, 1)
    omega = 1.0 / (10000.0 ** (jnp.arange(0, d, 2, dtype=jnp.float32) / d))
    angles = omega[None, :] * jnp.arange(s, dtype=jnp.float32)[:, None]
    imag = jnp.repeat(jnp.sin(angles), 2, axis=-1)
    imag = jnp.broadcast_to(imag[None, :, :], (b, s, d))
    return jax.lax.complex(real, imag)
